# baseline (speedup 1.0000x reference)
import jax
import jax.numpy as jnp
from jax import lax
from jax.experimental import pallas as pl
from jax.experimental.pallas import tpu as pltpu
from jax.experimental.pallas import tpu_sc as plsc

B, S, D = 4, 4096, 1024
NC, NS = 2, 16
NW = NC * NS
ROWS_PER_W = S // NW
CH = 4
CHE = 16
UPC = CHE // CH
NU = ROWS_PER_W // CH
NB = 5
NE = 2
UNROLL = 8
LPR = D // 16


def _add_unit(buf, emb, eoff):
    @plsc.parallel_loop(0, B * CH * LPR, step=1, unroll=UNROLL)
    def _(i):
        b = i // (CH * LPR)
        r = (i // LPR) % CH
        o = (i % LPR) * 16
        plsc.addupdate(buf.at[b, r, pl.ds(o, 16)], emb[eoff + r, pl.ds(o, 16)])


def _pos_add_sc(in_hbm, emb_hbm, out_hbm, *scratch):
    embs = scratch[:NE]
    bufs = scratch[NE:NE + NB]
    sems_e = scratch[NE + NB:NE + NB + NE]
    sems_i = scratch[NE + NB + NE:NE + NB + NE + NB]
    sems_o = scratch[NE + NB + NE + NB:]

    wid = lax.axis_index("s") * NC + lax.axis_index("c")
    base = wid * ROWS_PER_W

    def start_emb(c):
        return pltpu.async_copy(
            emb_hbm.at[pl.ds(base + c * CHE, CHE)], embs[c % NE], sems_e[c % NE])

    def start_in(u):
        return pltpu.async_copy(in_hbm.at[:, pl.ds(base + u * CH, CH)],
                                bufs[u % NB], sems_i[u % NB])

    def start_out(u):
        return pltpu.async_copy(bufs[u % NB],
                                out_hbm.at[:, pl.ds(base + u * CH, CH)],
                                sems_o[u % NB])

    nchunk = NU // UPC
    e_desc = {c: start_emb(c) for c in range(min(NE, nchunk))}
    i_desc = {u: start_in(u) for u in range(min(NB - 1, NU))}
    o_desc = {}
    o_waited = set()

    for u in range(NU):
        c, j = divmod(u, UPC)
        if j == 0:
            e_desc[c].wait()
        i_desc[u].wait()
        _add_unit(bufs[u % NB], embs[c % NE], j * CH)
        o_desc[u] = start_out(u)
        v = u + NB - 1
        if v < NU:
            if u >= 1:
                o_desc[u - 1].wait()
                o_waited.add(u - 1)
            i_desc[v] = start_in(v)
        if j == UPC - 1 and c + NE < nchunk:
            e_desc[c + NE] = start_emb(c + NE)

    for u in range(NU):
        if u not in o_waited:
            o_desc[u].wait()


@jax.jit
def _pos_add(inputs, emb):
    return pl.kernel(
        _pos_add_sc,
        out_type=jax.ShapeDtypeStruct((B, S, D), jnp.float32),
        mesh=plsc.VectorSubcoreMesh(core_axis_name="c", subcore_axis_name="s"),
        scratch_types=(
            [pltpu.VMEM((CHE, D), jnp.float32) for _ in range(NE)]
            + [pltpu.VMEM((B, CH, D), jnp.float32) for _ in range(NB)]
            + [pltpu.SemaphoreType.DMA for _ in range(NE + NB + NB)]
        ),
    )(inputs, emb)


def kernel(inputs, embedding):
    s = inputs.shape[1]
    return _pos_add(inputs, embedding[:s])
